# baseline (speedup 1.0000x reference)
import functools

import jax
import jax.numpy as jnp
from jax import lax
from jax.experimental import pallas as pl
from jax.experimental.pallas import tpu as pltpu
from jax.experimental.pallas import tpu_sc as plsc

N = 10000
E = 320000
H = 128
G = 64
NC = 2
NS = 16
NT = NC * NS
CH = 128
NCHUNK = -(-E // (NT * CH))
EPAD = NT * NCHUNK * CH
NCH0 = 112
NCH1 = 45
EPAD2 = NS * (NCH0 + NCH1) * CH
ACC_R = 10112
RPT = ACC_R // NS
TRASH = N

_mesh = plsc.VectorSubcoreMesh(core_axis_name="c", subcore_axis_name="s")


@functools.partial(
    pl.kernel,
    mesh=_mesh,
    out_type=jax.ShapeDtypeStruct((NC, ACC_R, H), jnp.float32),
    scratch_types=[
        pltpu.VMEM((NCHUNK, CH), jnp.int32),
        pltpu.VMEM((CH, H), jnp.float32),
        pltpu.VMEM_SHARED((ACC_R, H), jnp.float32),
        pltpu.SemaphoreType.DMA,
    ],
)
def _sc_degree(dst_hbm, zero_hbm, ones_hbm, out_hbm, idx_v, ones_v, acc_sh, sem):
    c = lax.axis_index("c")
    s = lax.axis_index("s")
    wid = c * NS + s
    r0 = pl.multiple_of(s * RPT, 8)
    pltpu.sync_copy(zero_hbm.at[pl.ds(r0, RPT)], acc_sh.at[pl.ds(r0, RPT)])
    pltpu.sync_copy(ones_hbm, ones_v)
    pltpu.sync_copy(dst_hbm.at[wid], idx_v)
    plsc.subcore_barrier()

    @pl.loop(0, NCHUNK)
    def _(i):
        pltpu.sync_copy(ones_v, acc_sh.at[idx_v.at[i]], add=True)

    plsc.subcore_barrier()
    pltpu.sync_copy(acc_sh.at[pl.ds(r0, RPT)], out_hbm.at[c, pl.ds(r0, RPT)])


@functools.partial(
    pl.kernel,
    mesh=_mesh,
    out_type=jax.ShapeDtypeStruct((NC, ACC_R, H), jnp.float32),
    scratch_types=[
        pltpu.VMEM((4, 2, CH), jnp.int32),
        pltpu.VMEM((CH, H), jnp.float32),
        pltpu.VMEM((CH, H), jnp.float32),
        pltpu.VMEM_SHARED((ACC_R, H), jnp.float32),
        pltpu.SemaphoreType.DMA,
        pltpu.SemaphoreType.DMA,
        pltpu.SemaphoreType.DMA,
        pltpu.SemaphoreType.DMA,
        pltpu.SemaphoreType.DMA,
        pltpu.SemaphoreType.DMA,
        pltpu.SemaphoreType.DMA,
        pltpu.SemaphoreType.DMA,
    ],
)
def _sc_msgpass(idx0_hbm, idx1_hbm, hs_hbm, zero_hbm, out_hbm,
                idx_v, rows_0, rows_1, acc_sh,
                si_0, si_1, si_2, si_3, sg_0, sg_1, ss_0, ss_1):
    c = lax.axis_index("c")
    s = lax.axis_index("s")
    r0 = pl.multiple_of(s * RPT, 8)
    pltpu.sync_copy(zero_hbm.at[pl.ds(r0, RPT)], acc_sh.at[pl.ds(r0, RPT)])
    plsc.subcore_barrier()

    SI = [si_0, si_1, si_2, si_3]
    SG = [sg_0, sg_1]
    SS = [ss_0, ss_1]
    R = [rows_0, rows_1]

    def _pipe(idx_hbm, nch):
        def idx_load(k, r):
            pltpu.async_copy(idx_hbm.at[s, k], idx_v.at[r], SI[r])

        def idx_wait(k, r):
            pltpu.make_async_copy(idx_hbm.at[s, k], idx_v.at[r], SI[r]).wait()

        def gather_start(r, r2):
            pltpu.async_copy(hs_hbm.at[idx_v.at[r, 0]], R[r2], SG[r2])

        def gather_wait(r, r2):
            pltpu.make_async_copy(hs_hbm.at[idx_v.at[r, 0]], R[r2], SG[r2]).wait()

        def scat_start(r, r2):
            pltpu.async_copy(R[r2], acc_sh.at[idx_v.at[r, 1]], SS[r2], add=True)

        def scat_wait(r, r2):
            pltpu.make_async_copy(R[r2], acc_sh.at[idx_v.at[r, 1]], SS[r2]).wait()

        def slot(k, r, first=False, load=True, nxt=True):
            r2 = r & 1
            q2 = 1 - r2
            if not first:
                scat_wait((r - 1) % 4, q2)
            if nxt:
                idx_wait(k + 1, (r + 1) % 4)
                gather_start((r + 1) % 4, q2)
            gather_wait(r, r2)
            scat_start(r, r2)
            if load:
                idx_load(k + 3, (r + 3) % 4)

        nb = nch // 4
        idx_load(0, 0)
        idx_load(1, 1)
        idx_load(2, 2)
        idx_wait(0, 0)
        gather_start(0, 0)
        slot(0, 0, first=True)
        slot(1, 1)
        slot(2, 2)
        slot(3, 3)

        @pl.loop(1, nb - 2)
        def _(b):
            k0 = b * 4
            slot(k0, 0)
            slot(k0 + 1, 1)
            slot(k0 + 2, 2)
            slot(k0 + 3, 3)

        for t in range(nch - 8, nch):
            slot(t, t % 4, load=(t + 3 < nch), nxt=(t + 1 < nch))
        scat_wait((nch - 1) % 4, (nch - 1) & 1)

    @pl.when(c == 1)
    def _():
        _pipe(idx0_hbm, NCH0)

    def _pipe_sync(idx_hbm, nch):
        idx_a = idx_v.at[0]
        idx_b = idx_v.at[1]
        pltpu.sync_copy(idx_hbm.at[s, 0], idx_a)
        pltpu.async_copy(hs_hbm.at[idx_a.at[0]], rows_0, sg_0)
        pltpu.sync_copy(idx_hbm.at[s, 1], idx_b)

        @pl.loop(0, (nch - 1) // 2)
        def _(j):
            i = j * 2
            pltpu.async_copy(hs_hbm.at[idx_b.at[0]], rows_1, sg_1)
            pltpu.make_async_copy(hs_hbm.at[idx_a.at[0]], rows_0, sg_0).wait()
            pltpu.sync_copy(rows_0, acc_sh.at[idx_a.at[1]], add=True)
            pltpu.sync_copy(idx_hbm.at[s, i + 2], idx_a)
            pltpu.async_copy(hs_hbm.at[idx_a.at[0]], rows_0, sg_0)
            pltpu.make_async_copy(hs_hbm.at[idx_b.at[0]], rows_1, sg_1).wait()
            pltpu.sync_copy(rows_1, acc_sh.at[idx_b.at[1]], add=True)
            pltpu.sync_copy(idx_hbm.at[s, jnp.minimum(i + 3, nch - 1)], idx_b)

        pltpu.make_async_copy(hs_hbm.at[idx_a.at[0]], rows_0, sg_0).wait()
        pltpu.sync_copy(rows_0, acc_sh.at[idx_a.at[1]], add=True)

    @pl.when(c == 0)
    def _():
        _pipe_sync(idx1_hbm, NCH1)

    plsc.subcore_barrier()
    pltpu.sync_copy(acc_sh.at[pl.ds(r0, RPT)], out_hbm.at[c, pl.ds(r0, RPT)])


def _tc_matmul(x_ref, w_ref, o_ref):
    o_ref[...] = jnp.dot(x_ref[...], w_ref[...],
                         preferred_element_type=jnp.float32,
                         precision=lax.Precision.HIGHEST)


def _tc_scale(degp_ref, h1_ref, hs_ref, dis_ref):
    deg = degp_ref[0, :N, 0:1] + degp_ref[1, :N, 0:1] + 1.0
    dis = lax.rsqrt(deg)
    dis_ref[...] = dis
    hs_ref[...] = h1_ref[...] * dis


def _tc_mid(p_ref, hs_ref, dis_ref, b_ref, w_ref, o_ref):
    t = p_ref[0, :N, :] + p_ref[1, :N, :] + hs_ref[...]
    h = jnp.maximum(t * dis_ref[...] + b_ref[...], 0.0)
    o_ref[...] = jnp.dot(h, w_ref[...],
                         preferred_element_type=jnp.float32) * dis_ref[...]


def _tc_final(p_ref, hs_ref, dis_ref, b_ref, batch_ref, o_ref):
    t = p_ref[0, :N, :] + p_ref[1, :N, :] + hs_ref[...]
    h = jnp.maximum(t * dis_ref[...] + b_ref[...], 0.0)
    gid = lax.broadcasted_iota(jnp.int32, (G, N), 0)
    oh = (batch_ref[...] == gid).astype(jnp.float32)
    sums = jnp.dot(oh.astype(jnp.bfloat16), h.astype(jnp.bfloat16),
                   preferred_element_type=jnp.float32)
    counts = jnp.sum(oh, axis=1, keepdims=True)
    o_ref[...] = sums / jnp.maximum(counts, 1.0)


def kernel(x, edge_index, batch, W1, b1, W2, b2):
    src = edge_index[0]
    dst = edge_index[1]
    padmax = max(EPAD, EPAD2) - E
    srcf = jnp.concatenate([src, jnp.zeros((padmax,), jnp.int32)])
    dstf = jnp.concatenate([dst, jnp.full((padmax,), TRASH, jnp.int32)])
    dstp = dstf[:EPAD].reshape(NT, NCHUNK, CH)
    n0 = NS * NCH0 * CH
    idx0 = jnp.stack([srcf[:n0].reshape(NS, NCH0, CH),
                      dstf[:n0].reshape(NS, NCH0, CH)], axis=2)
    idx1 = jnp.stack([srcf[n0:EPAD2].reshape(NS, NCH1, CH),
                      dstf[n0:EPAD2].reshape(NS, NCH1, CH)], axis=2)
    zeros_h = jnp.zeros((ACC_R, H), jnp.float32)
    ones_h = jnp.ones((CH, H), jnp.float32)
    batch2 = batch.reshape(1, N)

    f32 = jnp.float32
    degp = _sc_degree(dstp, zeros_h, ones_h)
    h1 = pl.pallas_call(
        _tc_matmul, out_shape=jax.ShapeDtypeStruct((N, H), f32))(x, W1)

    hs1, dis = pl.pallas_call(
        _tc_scale,
        out_shape=(jax.ShapeDtypeStruct((N, H), f32),
                   jax.ShapeDtypeStruct((N, 1), f32)))(degp, h1)

    p1 = _sc_msgpass(idx0, idx1, hs1, zeros_h)

    hs2 = pl.pallas_call(
        _tc_mid, out_shape=jax.ShapeDtypeStruct((N, H), f32))(p1, hs1, dis, b1, W2)

    p2 = _sc_msgpass(idx0, idx1, hs2, zeros_h)

    out = pl.pallas_call(
        _tc_final, out_shape=jax.ShapeDtypeStruct((G, H), f32))(p2, hs2, dis, b2, batch2)
    return out


# speedup vs baseline: 1.0014x; 1.0014x over previous
import functools

import jax
import jax.numpy as jnp
from jax import lax
from jax.experimental import pallas as pl
from jax.experimental.pallas import tpu as pltpu
from jax.experimental.pallas import tpu_sc as plsc

N = 10000
E = 320000
H = 128
G = 64
NC = 2
NS = 16
NT = NC * NS
CH = 128
NCHUNK = -(-E // (NT * CH))
EPAD = NT * NCHUNK * CH
NCH0 = 112
NCH1 = 45
EPAD2 = NS * (NCH0 + NCH1) * CH
ACC_R = 10112
RPT = ACC_R // NS
TRASH = N

_mesh = plsc.VectorSubcoreMesh(core_axis_name="c", subcore_axis_name="s")


@functools.partial(
    pl.kernel,
    mesh=_mesh,
    out_type=jax.ShapeDtypeStruct((NC, ACC_R, H), jnp.float32),
    scratch_types=[
        pltpu.VMEM((NCHUNK, CH), jnp.int32),
        pltpu.VMEM((CH, H), jnp.float32),
        pltpu.VMEM_SHARED((ACC_R, H), jnp.float32),
        pltpu.SemaphoreType.DMA,
    ],
)
def _sc_degree(dst_hbm, zero_hbm, ones_hbm, out_hbm, idx_v, ones_v, acc_sh, sem):
    c = lax.axis_index("c")
    s = lax.axis_index("s")
    wid = c * NS + s
    r0 = pl.multiple_of(s * RPT, 8)
    pltpu.sync_copy(zero_hbm.at[pl.ds(r0, RPT)], acc_sh.at[pl.ds(r0, RPT)])
    pltpu.sync_copy(ones_hbm, ones_v)
    pltpu.sync_copy(dst_hbm.at[wid], idx_v)
    plsc.subcore_barrier()

    @pl.loop(0, NCHUNK)
    def _(i):
        pltpu.sync_copy(ones_v, acc_sh.at[idx_v.at[i]], add=True)

    plsc.subcore_barrier()
    pltpu.sync_copy(acc_sh.at[pl.ds(r0, RPT)], out_hbm.at[c, pl.ds(r0, RPT)])


@functools.partial(
    pl.kernel,
    mesh=_mesh,
    out_type=jax.ShapeDtypeStruct((NC, ACC_R, H), jnp.float32),
    scratch_types=[
        pltpu.VMEM((4, 2, CH), jnp.int32),
        pltpu.VMEM((CH, H), jnp.float32),
        pltpu.VMEM((CH, H), jnp.float32),
        pltpu.VMEM_SHARED((ACC_R, H), jnp.float32),
        pltpu.SemaphoreType.DMA,
        pltpu.SemaphoreType.DMA,
        pltpu.SemaphoreType.DMA,
        pltpu.SemaphoreType.DMA,
        pltpu.SemaphoreType.DMA,
        pltpu.SemaphoreType.DMA,
        pltpu.SemaphoreType.DMA,
        pltpu.SemaphoreType.DMA,
    ],
)
def _sc_msgpass(idx0_hbm, idx1_hbm, hs_hbm, zero_hbm, out_hbm,
                idx_v, rows_0, rows_1, acc_sh,
                si_0, si_1, si_2, si_3, sg_0, sg_1, ss_0, ss_1):
    c = lax.axis_index("c")
    s = lax.axis_index("s")
    r0 = pl.multiple_of(s * RPT, 8)
    pltpu.sync_copy(zero_hbm.at[pl.ds(r0, RPT)], acc_sh.at[pl.ds(r0, RPT)])
    plsc.subcore_barrier()

    SI = [si_0, si_1, si_2, si_3]
    SG = [sg_0, sg_1]
    SS = [ss_0, ss_1]
    R = [rows_0, rows_1]

    def _pipe(idx_hbm, nch):
        def idx_load(k, r):
            pltpu.async_copy(idx_hbm.at[s, k], idx_v.at[r], SI[r])

        def idx_wait(k, r):
            pltpu.make_async_copy(idx_hbm.at[s, k], idx_v.at[r], SI[r]).wait()

        def gather_start(r, r2):
            pltpu.async_copy(hs_hbm.at[idx_v.at[r, 0]], R[r2], SG[r2])

        def gather_wait(r, r2):
            pltpu.make_async_copy(hs_hbm.at[idx_v.at[r, 0]], R[r2], SG[r2]).wait()

        def scat_start(r, r2):
            pltpu.async_copy(R[r2], acc_sh.at[idx_v.at[r, 1]], SS[r2], add=True)

        def scat_wait(r, r2):
            pltpu.make_async_copy(R[r2], acc_sh.at[idx_v.at[r, 1]], SS[r2]).wait()

        def slot(k, r, first=False, load=True, nxt=True):
            r2 = r & 1
            q2 = 1 - r2
            if not first:
                scat_wait((r - 1) % 4, q2)
            if nxt:
                idx_wait(k + 1, (r + 1) % 4)
                gather_start((r + 1) % 4, q2)
            gather_wait(r, r2)
            scat_start(r, r2)
            if load:
                idx_load(k + 3, (r + 3) % 4)

        nb = nch // 4
        idx_load(0, 0)
        idx_load(1, 1)
        idx_load(2, 2)
        idx_wait(0, 0)
        gather_start(0, 0)
        slot(0, 0, first=True)
        slot(1, 1)
        slot(2, 2)
        slot(3, 3)

        @pl.loop(1, nb - 2)
        def _(b):
            k0 = b * 4
            slot(k0, 0)
            slot(k0 + 1, 1)
            slot(k0 + 2, 2)
            slot(k0 + 3, 3)

        for t in range(nch - 8, nch):
            slot(t, t % 4, load=(t + 3 < nch), nxt=(t + 1 < nch))
        scat_wait((nch - 1) % 4, (nch - 1) & 1)

    @pl.when(c == 0)
    def _():
        _pipe(idx0_hbm, NCH0)

    def _pipe_sync(idx_hbm, nch):
        idx_a = idx_v.at[0]
        idx_b = idx_v.at[1]
        pltpu.sync_copy(idx_hbm.at[s, 0], idx_a)
        pltpu.async_copy(hs_hbm.at[idx_a.at[0]], rows_0, sg_0)
        pltpu.sync_copy(idx_hbm.at[s, 1], idx_b)

        @pl.loop(0, (nch - 1) // 2)
        def _(j):
            i = j * 2
            pltpu.async_copy(hs_hbm.at[idx_b.at[0]], rows_1, sg_1)
            pltpu.make_async_copy(hs_hbm.at[idx_a.at[0]], rows_0, sg_0).wait()
            pltpu.sync_copy(rows_0, acc_sh.at[idx_a.at[1]], add=True)
            pltpu.sync_copy(idx_hbm.at[s, i + 2], idx_a)
            pltpu.async_copy(hs_hbm.at[idx_a.at[0]], rows_0, sg_0)
            pltpu.make_async_copy(hs_hbm.at[idx_b.at[0]], rows_1, sg_1).wait()
            pltpu.sync_copy(rows_1, acc_sh.at[idx_b.at[1]], add=True)
            pltpu.sync_copy(idx_hbm.at[s, jnp.minimum(i + 3, nch - 1)], idx_b)

        pltpu.make_async_copy(hs_hbm.at[idx_a.at[0]], rows_0, sg_0).wait()
        pltpu.sync_copy(rows_0, acc_sh.at[idx_a.at[1]], add=True)

    @pl.when(c == 1)
    def _():
        _pipe_sync(idx1_hbm, NCH1)

    plsc.subcore_barrier()
    pltpu.sync_copy(acc_sh.at[pl.ds(r0, RPT)], out_hbm.at[c, pl.ds(r0, RPT)])


def _tc_matmul(x_ref, w_ref, o_ref):
    o_ref[...] = jnp.dot(x_ref[...], w_ref[...],
                         preferred_element_type=jnp.float32,
                         precision=lax.Precision.HIGHEST)


def _tc_scale(degp_ref, h1_ref, hs_ref, dis_ref):
    deg = degp_ref[0, :N, 0:1] + degp_ref[1, :N, 0:1] + 1.0
    dis = lax.rsqrt(deg)
    dis_ref[...] = dis
    hs_ref[...] = h1_ref[...] * dis


def _tc_mid(p_ref, hs_ref, dis_ref, b_ref, w_ref, o_ref):
    t = p_ref[0, :N, :] + p_ref[1, :N, :] + hs_ref[...]
    h = jnp.maximum(t * dis_ref[...] + b_ref[...], 0.0)
    o_ref[...] = jnp.dot(h, w_ref[...],
                         preferred_element_type=jnp.float32) * dis_ref[...]


def _tc_final(p_ref, hs_ref, dis_ref, b_ref, batch_ref, o_ref):
    t = p_ref[0, :N, :] + p_ref[1, :N, :] + hs_ref[...]
    h = jnp.maximum(t * dis_ref[...] + b_ref[...], 0.0)
    gid = lax.broadcasted_iota(jnp.int32, (G, N), 0)
    oh = (batch_ref[...] == gid).astype(jnp.float32)
    sums = jnp.dot(oh.astype(jnp.bfloat16), h.astype(jnp.bfloat16),
                   preferred_element_type=jnp.float32)
    counts = jnp.sum(oh, axis=1, keepdims=True)
    o_ref[...] = sums / jnp.maximum(counts, 1.0)


def kernel(x, edge_index, batch, W1, b1, W2, b2):
    src = edge_index[0]
    dst = edge_index[1]
    padmax = max(EPAD, EPAD2) - E
    srcf = jnp.concatenate([src, jnp.zeros((padmax,), jnp.int32)])
    dstf = jnp.concatenate([dst, jnp.full((padmax,), TRASH, jnp.int32)])
    dstp = dstf[:EPAD].reshape(NT, NCHUNK, CH)
    n0 = NS * NCH0 * CH
    idx0 = jnp.stack([srcf[:n0].reshape(NS, NCH0, CH),
                      dstf[:n0].reshape(NS, NCH0, CH)], axis=2)
    idx1 = jnp.stack([srcf[n0:EPAD2].reshape(NS, NCH1, CH),
                      dstf[n0:EPAD2].reshape(NS, NCH1, CH)], axis=2)
    zeros_h = jnp.zeros((ACC_R, H), jnp.float32)
    ones_h = jnp.ones((CH, H), jnp.float32)
    batch2 = batch.reshape(1, N)

    f32 = jnp.float32
    degp = _sc_degree(dstp, zeros_h, ones_h)
    h1 = pl.pallas_call(
        _tc_matmul, out_shape=jax.ShapeDtypeStruct((N, H), f32))(x, W1)

    hs1, dis = pl.pallas_call(
        _tc_scale,
        out_shape=(jax.ShapeDtypeStruct((N, H), f32),
                   jax.ShapeDtypeStruct((N, 1), f32)))(degp, h1)

    p1 = _sc_msgpass(idx0, idx1, hs1, zeros_h)

    hs2 = pl.pallas_call(
        _tc_mid, out_shape=jax.ShapeDtypeStruct((N, H), f32))(p1, hs1, dis, b1, W2)

    p2 = _sc_msgpass(idx0, idx1, hs2, zeros_h)

    out = pl.pallas_call(
        _tc_final, out_shape=jax.ShapeDtypeStruct((G, H), f32))(p2, hs2, dis, b2, batch2)
    return out
